# Initial kernel scaffold; baseline (speedup 1.0000x reference)
import jax
import jax.numpy as jnp
from jax.experimental import pallas as pl


def kernel(feat, segment_ids, Wg, bg, Wn, bn):
    raise NotImplementedError("write your pallas kernel here")


# speedup vs baseline: 9.9249x; 9.9249x over previous
import functools
import jax
import jax.numpy as jnp
from jax import lax
from jax.experimental import pallas as pl
from jax.experimental.pallas import tpu as pltpu

N = 50000
D = 512
G = 256
B = 2000
NB = N // B

_NEG = -1e30


def _gate_kernel(feat, seg, wg, bg, gate, m_out, cnt_out):
    i = pl.program_id(0)

    @pl.when(i == 0)
    def _():
        m_out[...] = jnp.full_like(m_out, _NEG)
        cnt_out[...] = jnp.zeros_like(cnt_out)

    x = feat[...]
    g = jnp.dot(x, wg[...], preferred_element_type=jnp.float32) + bg[0, 0]
    gate[...] = g
    s = seg[0, 0, :]
    ids = lax.broadcasted_iota(jnp.int32, (B, G), 1)
    oh = s[:, None] == ids
    lm = jnp.max(jnp.where(oh, g, _NEG), axis=0)
    m_out[0, :] = jnp.maximum(m_out[0, :], lm)
    cnt_out[0, :] = cnt_out[0, :] + jnp.sum(oh.astype(jnp.float32), axis=0)


def _pexp_kernel(gate, seg, m, p_out, den_out):
    i = pl.program_id(0)

    @pl.when(i == 0)
    def _():
        den_out[...] = jnp.zeros_like(den_out)

    g = gate[...]
    s = seg[0, 0, :]
    ids = lax.broadcasted_iota(jnp.int32, (B, G), 1)
    oh = s[:, None] == ids
    m_sel = jnp.sum(jnp.where(oh, m[0, :][None, :], 0.0), axis=1)
    pv = jnp.exp(g[:, 0] - m_sel)
    p_out[...] = pv[:, None]
    den_out[0, :] = den_out[0, :] + jnp.sum(jnp.where(oh, pv[:, None], 0.0), axis=0)


def _pool_kernel(feat, seg, p, pooled):
    i = pl.program_id(0)

    @pl.when(i == 0)
    def _():
        pooled[...] = jnp.zeros_like(pooled)

    x = feat[...]
    w = x * p[...]
    s = seg[0, 0, :]
    ids = lax.broadcasted_iota(jnp.int32, (B, G), 1)
    oh = (s[:, None] == ids).astype(jnp.float32)
    pooled[...] = pooled[...] + lax.dot_general(
        oh, w, dimension_numbers=(((0,), (0,)), ((), ())),
        preferred_element_type=jnp.float32)


def _final_kernel(pooled, den, cnt, wn, bn, out):
    d = den[0, :][:, None]
    msk = cnt[0, :][:, None] > 0.5
    inv = jnp.where(d > 0, 1.0 / jnp.where(d > 0, d, 1.0), 0.0)
    pn = pooled[...] * inv
    out[...] = jnp.dot(pn, wn[...], preferred_element_type=jnp.float32) + \
        jnp.where(msk, bn[...], 0.0)


def kernel(feat, segment_ids, Wg, bg, Wn, bn):
    seg3 = segment_ids.astype(jnp.int32).reshape(NB, 1, B)
    bg2 = bg.reshape(1, 1)
    bn2 = bn.reshape(1, D)

    gate, m, cnt = pl.pallas_call(
        _gate_kernel,
        grid=(NB,),
        in_specs=[
            pl.BlockSpec((B, D), lambda i: (i, 0)),
            pl.BlockSpec((1, 1, B), lambda i: (i, 0, 0)),
            pl.BlockSpec((D, 1), lambda i: (0, 0)),
            pl.BlockSpec((1, 1), lambda i: (0, 0)),
        ],
        out_specs=[
            pl.BlockSpec((B, 1), lambda i: (i, 0)),
            pl.BlockSpec((1, G), lambda i: (0, 0)),
            pl.BlockSpec((1, G), lambda i: (0, 0)),
        ],
        out_shape=[
            jax.ShapeDtypeStruct((N, 1), jnp.float32),
            jax.ShapeDtypeStruct((1, G), jnp.float32),
            jax.ShapeDtypeStruct((1, G), jnp.float32),
        ],
    )(feat, seg3, Wg, bg2)

    p, den = pl.pallas_call(
        _pexp_kernel,
        grid=(NB,),
        in_specs=[
            pl.BlockSpec((B, 1), lambda i: (i, 0)),
            pl.BlockSpec((1, 1, B), lambda i: (i, 0, 0)),
            pl.BlockSpec((1, G), lambda i: (0, 0)),
        ],
        out_specs=[
            pl.BlockSpec((B, 1), lambda i: (i, 0)),
            pl.BlockSpec((1, G), lambda i: (0, 0)),
        ],
        out_shape=[
            jax.ShapeDtypeStruct((N, 1), jnp.float32),
            jax.ShapeDtypeStruct((1, G), jnp.float32),
        ],
    )(gate, seg3, m)

    pooled = pl.pallas_call(
        _pool_kernel,
        grid=(NB,),
        in_specs=[
            pl.BlockSpec((B, D), lambda i: (i, 0)),
            pl.BlockSpec((1, 1, B), lambda i: (i, 0, 0)),
            pl.BlockSpec((B, 1), lambda i: (i, 0)),
        ],
        out_specs=pl.BlockSpec((G, D), lambda i: (0, 0)),
        out_shape=jax.ShapeDtypeStruct((G, D), jnp.float32),
    )(feat, seg3, p)

    out = pl.pallas_call(
        _final_kernel,
        in_specs=[
            pl.BlockSpec((G, D), lambda: (0, 0)),
            pl.BlockSpec((1, G), lambda: (0, 0)),
            pl.BlockSpec((1, G), lambda: (0, 0)),
            pl.BlockSpec((D, D), lambda: (0, 0)),
            pl.BlockSpec((1, D), lambda: (0, 0)),
        ],
        out_specs=pl.BlockSpec((G, D), lambda: (0, 0)),
        out_shape=jax.ShapeDtypeStruct((G, D), jnp.float32),
    )(pooled, den, cnt, Wn, bn2)

    return out
